# Initial kernel scaffold; baseline (speedup 1.0000x reference)
import jax
import jax.numpy as jnp
from jax.experimental import pallas as pl


def kernel(x, table):
    raise NotImplementedError("write your pallas kernel here")


# speedup vs baseline: 5.2313x; 5.2313x over previous
import functools

import jax
import jax.numpy as jnp
from jax import lax
from jax.experimental import pallas as pl
from jax.experimental.pallas import tpu as pltpu
from jax.experimental.pallas import tpu_sc as plsc

NUM_CORES = 2
NUM_SUBCORES = 16
NUM_WORKERS = NUM_CORES * NUM_SUBCORES


@functools.lru_cache(maxsize=None)
def _make_gather(B, V, D, chunk):
    b_per_w = B // NUM_WORKERS
    n_chunks = b_per_w // chunk
    mesh = plsc.VectorSubcoreMesh(core_axis_name="c", subcore_axis_name="s")

    @functools.partial(
        pl.kernel,
        mesh=mesh,
        out_type=jax.ShapeDtypeStruct((B, D), jnp.float32),
        scratch_types=[
            pltpu.VMEM((chunk,), jnp.int32),
            pltpu.VMEM((chunk, D), jnp.float32),
            pltpu.SemaphoreType.DMA,
        ],
        compiler_params=pltpu.CompilerParams(use_tc_tiling_on_sc=False),
    )
    def gather_kernel(idx_hbm, table_hbm, out_hbm, idx_v, rows_v, sem):
        wid = lax.axis_index("s") * NUM_CORES + lax.axis_index("c")
        base = wid * b_per_w

        def body(i, _):
            off = base + i * chunk
            pltpu.sync_copy(idx_hbm.at[pl.ds(off, chunk)], idx_v)
            pltpu.async_copy(table_hbm.at[idx_v], rows_v, sem).wait()
            pltpu.sync_copy(rows_v, out_hbm.at[pl.ds(off, chunk)])
            return 0

        lax.fori_loop(0, n_chunks, body, 0)

    return gather_kernel


def kernel(x, table):
    B0, S = x.shape
    V, D = table.shape
    B = B0 * S
    idx = x.reshape(B).astype(jnp.int32)
    out = _make_gather(B, V, D, 2560)(idx, table)
    return out.reshape(B0, S, D)


# speedup vs baseline: 5.2930x; 1.0118x over previous
import functools

import jax
import jax.numpy as jnp
from jax import lax
from jax.experimental import pallas as pl
from jax.experimental.pallas import tpu as pltpu
from jax.experimental.pallas import tpu_sc as plsc

NUM_CORES = 2
NUM_SUBCORES = 16
NUM_WORKERS = NUM_CORES * NUM_SUBCORES
NBUF = 3


@functools.lru_cache(maxsize=None)
def _make_gather(B, V, D, chunk):
    b_per_w = B // NUM_WORKERS
    n_chunks = b_per_w // chunk
    mesh = plsc.VectorSubcoreMesh(core_axis_name="c", subcore_axis_name="s")

    @functools.partial(
        pl.kernel,
        mesh=mesh,
        out_type=jax.ShapeDtypeStruct((B, D), jnp.float32),
        scratch_types=[
            pltpu.VMEM((b_per_w,), jnp.int32),
            [pltpu.VMEM((chunk, D), jnp.float32) for _ in range(NBUF)],
            [pltpu.SemaphoreType.DMA for _ in range(NBUF)],
            [pltpu.SemaphoreType.DMA for _ in range(NBUF)],
        ],
        compiler_params=pltpu.CompilerParams(use_tc_tiling_on_sc=False),
    )
    def gather_kernel(idx_hbm, table_hbm, out_hbm, idx_v, rows, gsem, ssem):
        wid = lax.axis_index("s") * NUM_CORES + lax.axis_index("c")
        base = wid * b_per_w
        pltpu.sync_copy(idx_hbm.at[pl.ds(base, b_per_w)], idx_v)

        def gather(g, b):
            pltpu.async_copy(
                table_hbm.at[idx_v.at[pl.ds(g * chunk, chunk)]], rows[b], gsem[b]
            )

        def store(g, b):
            return pltpu.make_async_copy(
                rows[b], out_hbm.at[pl.ds(base + g * chunk, chunk)], ssem[b]
            )

        for g in range(n_chunks):
            b = g % NBUF
            if g >= NBUF:
                store(g - NBUF, b).wait()
            gather(g, b)
            if g >= 1:
                pb = (g - 1) % NBUF
                pltpu.make_async_copy(
                    table_hbm.at[idx_v.at[pl.ds((g - 1) * chunk, chunk)]],
                    rows[pb],
                    gsem[pb],
                ).wait()
                store(g - 1, pb).start()
        b = (n_chunks - 1) % NBUF
        pltpu.make_async_copy(
            table_hbm.at[idx_v.at[pl.ds((n_chunks - 1) * chunk, chunk)]],
            rows[b],
            gsem[b],
        ).wait()
        store(n_chunks - 1, b).start()
        for g in range(n_chunks - NBUF, n_chunks):
            store(g, g % NBUF).wait()

    return gather_kernel


def kernel(x, table):
    B0, S = x.shape
    V, D = table.shape
    B = B0 * S
    idx = x.reshape(B).astype(jnp.int32)
    out = _make_gather(B, V, D, 1024)(idx, table)
    return out.reshape(B0, S, D)


# speedup vs baseline: 10.4520x; 1.9747x over previous
import functools

import jax
import jax.numpy as jnp
from jax import lax
from jax.experimental import pallas as pl
from jax.experimental.pallas import tpu as pltpu
from jax.experimental.pallas import tpu_sc as plsc

NUM_CORES = 2
NUM_SUBCORES = 16
L = 16


@functools.lru_cache(maxsize=None)
def _make_gather(S, B, V, D):
    n_bhi = B // 128
    n_grp = B // L
    mesh = plsc.VectorSubcoreMesh(core_axis_name="c", subcore_axis_name="s")

    @functools.partial(
        pl.kernel,
        mesh=mesh,
        out_type=jax.ShapeDtypeStruct((S, D // 8, n_bhi, 8, 128), jnp.float32),
        scratch_types=[
            pltpu.VMEM((V,), jnp.float32),
            [pltpu.VMEM((B,), jnp.int32) for _ in range(2)],
            [pltpu.VMEM((n_bhi, 128), jnp.float32) for _ in range(2)],
            [pltpu.SemaphoreType.DMA for _ in range(2)],
            [pltpu.SemaphoreType.DMA for _ in range(2)],
        ],
        compiler_params=pltpu.CompilerParams(
            use_tc_tiling_on_sc=False, needs_layout_passes=False
        ),
    )
    def gather_kernel(xT_hbm, tabT_hbm, out_hbm, trow, idx2, stage2, isem, osem):
        wid = lax.axis_index("s") * NUM_CORES + lax.axis_index("c")
        c_hi = wid // 8
        c_lo = wid % 8
        pltpu.sync_copy(tabT_hbm.at[wid], trow)
        pltpu.async_copy(xT_hbm.at[0], idx2[0], isem[0])

        def step(s, par):
            idx_v, stage_v = idx2[par], stage2[par]
            pltpu.make_async_copy(xT_hbm.at[s], idx_v, isem[par]).wait()

            @pl.when(s + 1 < S)
            def _prefetch():
                pltpu.async_copy(xT_hbm.at[s + 1], idx2[1 - par], isem[1 - par])

            @pl.when(s >= 2)
            def _drain():
                pltpu.make_async_copy(
                    stage_v, out_hbm.at[s - 2, c_hi, :, c_lo, :], osem[par]
                ).wait()

            def vgather(i, _):
                for k in range(16):
                    g16 = (i * 16 + k) * L
                    idx16 = idx_v[pl.ds(g16, L)]
                    vals = plsc.load_gather(trow, [idx16])
                    stage_v[2 * i + k // 8, pl.ds((k % 8) * L, L)] = vals
                return 0

            lax.fori_loop(0, n_grp // 16, vgather, 0)
            pltpu.async_copy(
                stage_v, out_hbm.at[s, c_hi, :, c_lo, :], osem[par]
            )

        def two(j, _):
            step(2 * j, 0)
            step(2 * j + 1, 1)
            return 0

        lax.fori_loop(0, S // 2, two, 0)
        for par, s in ((0, S - 2), (1, S - 1)):
            pltpu.make_async_copy(
                stage2[par], out_hbm.at[s, c_hi, :, c_lo, :], osem[par]
            ).wait()

    return gather_kernel


def kernel(x, table):
    B, S = x.shape
    V, D = table.shape
    xT = x.T.astype(jnp.int32)
    tabT = table.T
    out5 = _make_gather(S, B, V, D)(xT, tabT)
    return out5.transpose(2, 4, 0, 1, 3).reshape(B, S, D)


# speedup vs baseline: 11.7334x; 1.1226x over previous
import functools

import jax
import jax.numpy as jnp
from jax import lax
from jax.experimental import pallas as pl
from jax.experimental.pallas import tpu as pltpu
from jax.experimental.pallas import tpu_sc as plsc

NUM_CORES = 2
NUM_SUBCORES = 16
L = 16


@functools.lru_cache(maxsize=None)
def _make_gather(S, B, V, D):
    n_bhi = B // 128
    n_grp = B // L
    mesh = plsc.VectorSubcoreMesh(core_axis_name="c", subcore_axis_name="s")

    @functools.partial(
        pl.kernel,
        mesh=mesh,
        out_type=jax.ShapeDtypeStruct((S, D // 8, n_bhi, 8, 128), jnp.float32),
        scratch_types=[
            pltpu.VMEM((V,), jnp.float32),
            [pltpu.VMEM((B,), jnp.int32) for _ in range(2)],
            [pltpu.VMEM((n_bhi, 128), jnp.float32) for _ in range(2)],
            [pltpu.SemaphoreType.DMA for _ in range(2)],
            [pltpu.SemaphoreType.DMA for _ in range(2)],
        ],
        compiler_params=pltpu.CompilerParams(
            use_tc_tiling_on_sc=False, needs_layout_passes=False
        ),
    )
    def gather_kernel(xT_hbm, tabT_hbm, out_hbm, trow, idx2, stage2, isem, osem):
        wid = lax.axis_index("s") * NUM_CORES + lax.axis_index("c")
        c_hi = wid // 8
        c_lo = wid % 8
        pltpu.sync_copy(tabT_hbm.at[wid], trow)
        pltpu.async_copy(xT_hbm.at[0], idx2[0], isem[0])

        def step(s, par):
            idx_v, stage_v = idx2[par], stage2[par]
            pltpu.make_async_copy(xT_hbm.at[s], idx_v, isem[par]).wait()

            @pl.when(s + 1 < S)
            def _prefetch():
                pltpu.async_copy(xT_hbm.at[s + 1], idx2[1 - par], isem[1 - par])

            @pl.when(s >= 2)
            def _drain():
                pltpu.make_async_copy(
                    stage_v, out_hbm.at[s - 2, c_hi, :, c_lo, :], osem[par]
                ).wait()

            @plsc.parallel_loop(0, n_grp, step=1, unroll=8)
            def _vg(g):
                idx16 = idx_v[pl.ds(g * L, L)]
                vals = plsc.load_gather(trow, [idx16])
                stage_v[g // 8, pl.ds((g % 8) * L, L)] = vals
            pltpu.async_copy(
                stage_v, out_hbm.at[s, c_hi, :, c_lo, :], osem[par]
            )

        def two(j, _):
            step(2 * j, 0)
            step(2 * j + 1, 1)
            return 0

        lax.fori_loop(0, S // 2, two, 0)
        for par, s in ((0, S - 2), (1, S - 1)):
            pltpu.make_async_copy(
                stage2[par], out_hbm.at[s, c_hi, :, c_lo, :], osem[par]
            ).wait()

    return gather_kernel


def kernel(x, table):
    B, S = x.shape
    V, D = table.shape
    xT = x.T.astype(jnp.int32)
    tabT = table.T
    out5 = _make_gather(S, B, V, D)(xT, tabT)
    return out5.transpose(2, 4, 0, 1, 3).reshape(B, S, D)


# speedup vs baseline: 13.5500x; 1.1548x over previous
import functools

import jax
import jax.numpy as jnp
from jax import lax
from jax.experimental import pallas as pl
from jax.experimental.pallas import tpu as pltpu
from jax.experimental.pallas import tpu_sc as plsc

NUM_CORES = 2
NUM_SUBCORES = 16
L = 16


@functools.lru_cache(maxsize=None)
def _make_gather(S, B, V, D):
    n_bhi = B // 128
    n_grp = B // L
    mesh = plsc.VectorSubcoreMesh(core_axis_name="c", subcore_axis_name="s")

    @functools.partial(
        pl.kernel,
        mesh=mesh,
        out_type=jax.ShapeDtypeStruct((S, D // 8, n_bhi, 8, 128), jnp.float32),
        scratch_types=[
            pltpu.VMEM((V,), jnp.float32),
            pltpu.VMEM_SHARED((2, B), jnp.int32),
            pltpu.VMEM((B,), jnp.int32),
            [pltpu.VMEM((n_bhi, 128), jnp.float32) for _ in range(2)],
            pltpu.SemaphoreType.DMA,
            [pltpu.SemaphoreType.DMA for _ in range(2)],
        ],
        compiler_params=pltpu.CompilerParams(
            use_tc_tiling_on_sc=False, needs_layout_passes=False
        ),
    )
    def gather_kernel(
        xT_hbm, tabT_hbm, out_hbm, trow, xsp, idx_v, stage2, fsem, osem
    ):
        sid = lax.axis_index("s")
        wid = sid * NUM_CORES + lax.axis_index("c")
        c_hi = wid // 8
        c_lo = wid % 8
        fetcher = sid == 0

        @pl.when(fetcher)
        def _fetch0():
            pltpu.async_copy(xT_hbm.at[0], xsp.at[0], fsem)

        pltpu.sync_copy(tabT_hbm.at[wid], trow)

        def step(s, par):
            stage_v = stage2[par]

            @pl.when(fetcher)
            def _fwait():
                pltpu.make_async_copy(xT_hbm.at[s], xsp.at[par], fsem).wait()

            plsc.subcore_barrier()

            @pl.when(fetcher & (s + 1 < S))
            def _prefetch():
                pltpu.async_copy(xT_hbm.at[s + 1], xsp.at[1 - par], fsem)

            pltpu.sync_copy(xsp.at[par], idx_v)

            @pl.when(s >= 2)
            def _drain():
                pltpu.make_async_copy(
                    stage_v, out_hbm.at[s - 2, c_hi, :, c_lo, :], osem[par]
                ).wait()

            @plsc.parallel_loop(0, n_grp, step=1, unroll=8)
            def _vg(g):
                idx16 = idx_v[pl.ds(g * L, L)]
                vals = plsc.load_gather(trow, [idx16])
                stage_v[g // 8, pl.ds((g % 8) * L, L)] = vals
            pltpu.async_copy(
                stage_v, out_hbm.at[s, c_hi, :, c_lo, :], osem[par]
            )

        def two(j, _):
            step(2 * j, 0)
            step(2 * j + 1, 1)
            return 0

        lax.fori_loop(0, S // 2, two, 0)
        for par, s in ((0, S - 2), (1, S - 1)):
            pltpu.make_async_copy(
                stage2[par], out_hbm.at[s, c_hi, :, c_lo, :], osem[par]
            ).wait()

    return gather_kernel


def kernel(x, table):
    B, S = x.shape
    V, D = table.shape
    xT = x.T.astype(jnp.int32)
    tabT = table.T
    out5 = _make_gather(S, B, V, D)(xT, tabT)
    return out5.transpose(2, 4, 0, 1, 3).reshape(B, S, D)


# speedup vs baseline: 13.7458x; 1.0144x over previous
import functools

import jax
import jax.numpy as jnp
from jax import lax
from jax.experimental import pallas as pl
from jax.experimental.pallas import tpu as pltpu
from jax.experimental.pallas import tpu_sc as plsc

NUM_CORES = 2
NUM_SUBCORES = 16
L = 16


@functools.lru_cache(maxsize=None)
def _make_gather(S, B, V, D):
    n_bhi = B // 128
    n_grp = B // L
    mesh = plsc.VectorSubcoreMesh(core_axis_name="c", subcore_axis_name="s")

    @functools.partial(
        pl.kernel,
        mesh=mesh,
        out_type=jax.ShapeDtypeStruct((S, D // 8, n_bhi, 8, 128), jnp.float32),
        scratch_types=[
            pltpu.VMEM((V,), jnp.float32),
            pltpu.VMEM_SHARED((2, B), jnp.int32),
            [pltpu.VMEM((B,), jnp.int32) for _ in range(2)],
            [pltpu.VMEM((n_bhi, 128), jnp.float32) for _ in range(2)],
            pltpu.SemaphoreType.DMA,
            [pltpu.SemaphoreType.DMA for _ in range(2)],
            [pltpu.SemaphoreType.DMA for _ in range(2)],
        ],
        compiler_params=pltpu.CompilerParams(
            use_tc_tiling_on_sc=False, needs_layout_passes=False
        ),
    )
    def gather_kernel(
        xT_hbm, tabT_hbm, out_hbm, trow, xsp, idx2, stage2, fsem, csem, osem
    ):
        sid = lax.axis_index("s")
        wid = sid * NUM_CORES + lax.axis_index("c")
        c_hi = wid // 8
        c_lo = wid % 8
        fetcher = sid == 0

        @pl.when(fetcher)
        def _fetch0():
            pltpu.make_async_copy(xT_hbm.at[0], xsp.at[0], fsem).start()
            pltpu.make_async_copy(xT_hbm.at[0], xsp.at[0], fsem).wait()

        pltpu.sync_copy(tabT_hbm.at[wid], trow)
        plsc.subcore_barrier()
        pltpu.async_copy(xsp.at[0], idx2[0], csem[0])

        @pl.when(fetcher)
        def _fetch1():
            pltpu.async_copy(xT_hbm.at[1], xsp.at[1], fsem)

        def step(s, par):
            idx_v, stage_v = idx2[par], stage2[par]
            nxt = 1 - par
            pltpu.make_async_copy(xsp.at[par], idx_v, csem[par]).wait()

            @pl.when(fetcher & (s + 1 < S))
            def _fwait():
                pltpu.make_async_copy(xT_hbm.at[s + 1], xsp.at[nxt], fsem).wait()

            plsc.subcore_barrier()

            @pl.when(fetcher & (s + 2 < S))
            def _prefetch():
                pltpu.async_copy(xT_hbm.at[s + 2], xsp.at[par], fsem)

            @pl.when(s + 1 < S)
            def _bcast():
                pltpu.async_copy(xsp.at[nxt], idx2[nxt], csem[nxt])

            @pl.when(s >= 2)
            def _drain():
                pltpu.make_async_copy(
                    stage_v, out_hbm.at[s - 2, c_hi, :, c_lo, :], osem[par]
                ).wait()

            @plsc.parallel_loop(0, n_grp, step=1, unroll=8)
            def _vg(g):
                idx16 = idx_v[pl.ds(g * L, L)]
                vals = plsc.load_gather(trow, [idx16])
                stage_v[g // 8, pl.ds((g % 8) * L, L)] = vals

            pltpu.async_copy(
                stage_v, out_hbm.at[s, c_hi, :, c_lo, :], osem[par]
            )

        def two(j, _):
            step(2 * j, 0)
            step(2 * j + 1, 1)
            return 0

        lax.fori_loop(0, S // 2, two, 0)
        for par, s in ((0, S - 2), (1, S - 1)):
            pltpu.make_async_copy(
                stage2[par], out_hbm.at[s, c_hi, :, c_lo, :], osem[par]
            ).wait()

    return gather_kernel


def kernel(x, table):
    B, S = x.shape
    V, D = table.shape
    xT = x.T.astype(jnp.int32)
    tabT = table.T
    out5 = _make_gather(S, B, V, D)(xT, tabT)
    return out5.transpose(2, 4, 0, 1, 3).reshape(B, S, D)


# speedup vs baseline: 17.4199x; 1.2673x over previous
import functools

import jax
import jax.numpy as jnp
from jax import lax
from jax.experimental import pallas as pl
from jax.experimental.pallas import tpu as pltpu
from jax.experimental.pallas import tpu_sc as plsc

NUM_CORES = 2
NUM_SUBCORES = 16
L = 16
NIDX = 4


@functools.lru_cache(maxsize=None)
def _make_gather(S, B, V, D):
    n_bhi = B // 128
    n_grp = B // L
    mesh = plsc.VectorSubcoreMesh(core_axis_name="c", subcore_axis_name="s")

    @functools.partial(
        pl.kernel,
        mesh=mesh,
        out_type=jax.ShapeDtypeStruct((S, D // 8, n_bhi, 8, 128), jnp.float32),
        scratch_types=[
            pltpu.VMEM((V,), jnp.float32),
            [pltpu.VMEM((B,), jnp.int32) for _ in range(NIDX)],
            [pltpu.VMEM((n_bhi, 128), jnp.float32) for _ in range(2)],
            [pltpu.SemaphoreType.DMA for _ in range(NIDX)],
            [pltpu.SemaphoreType.DMA for _ in range(2)],
        ],
        compiler_params=pltpu.CompilerParams(
            use_tc_tiling_on_sc=False, needs_layout_passes=False
        ),
    )
    def gather_kernel(xT_hbm, tabT_hbm, out_hbm, trow, idx4, stage2, isem, osem):
        wid = lax.axis_index("s") * NUM_CORES + lax.axis_index("c")
        c_hi = wid // 8
        c_lo = wid % 8
        for r in range(NIDX - 1):
            pltpu.async_copy(xT_hbm.at[r], idx4[r], isem[r])
        pltpu.sync_copy(tabT_hbm.at[wid], trow)

        def step(s, r, par):
            idx_v, stage_v = idx4[r], stage2[par]
            pltpu.make_async_copy(xT_hbm.at[s], idx_v, isem[r]).wait()

            @pl.when(s + NIDX - 1 < S)
            def _prefetch():
                pltpu.async_copy(
                    xT_hbm.at[s + NIDX - 1],
                    idx4[(r + NIDX - 1) % NIDX],
                    isem[(r + NIDX - 1) % NIDX],
                )

            @pl.when(s >= 2)
            def _drain():
                pltpu.make_async_copy(
                    stage_v, out_hbm.at[s - 2, c_hi, :, c_lo, :], osem[par]
                ).wait()

            @plsc.parallel_loop(0, n_grp, step=1, unroll=8)
            def _vg(g):
                idx16 = idx_v[pl.ds(g * L, L)]
                vals = plsc.load_gather(trow, [idx16])
                stage_v[g // 8, pl.ds((g % 8) * L, L)] = vals

            pltpu.async_copy(
                stage_v, out_hbm.at[s, c_hi, :, c_lo, :], osem[par]
            )

        def four(j, _):
            for q in range(NIDX):
                step(NIDX * j + q, q, q % 2)
            return 0

        lax.fori_loop(0, S // NIDX, four, 0)
        for par, s in ((0, S - 2), (1, S - 1)):
            pltpu.make_async_copy(
                stage2[par], out_hbm.at[s, c_hi, :, c_lo, :], osem[par]
            ).wait()

    return gather_kernel


def kernel(x, table):
    B, S = x.shape
    V, D = table.shape
    xT = x.T.astype(jnp.int32)
    tabT = table.T
    out5 = _make_gather(S, B, V, D)(xT, tabT)
    return out5.transpose(2, 4, 0, 1, 3).reshape(B, S, D)
